# Initial kernel scaffold; baseline (speedup 1.0000x reference)
import jax
import jax.numpy as jnp
from jax.experimental import pallas as pl


def kernel(x, edge_index, edge_attr, W_ne, b_ne, W_ee, b_ee, We1, be1, We2, be2, root_e1, bias_e1, root_e2, bias_e2, Wd1, bd1, Wd2, bd2, root_d1, bias_d1, root_d2, bias_d2, W_nt, b_nt):
    raise NotImplementedError("write your pallas kernel here")


# speedup vs baseline: 1.8784x; 1.8784x over previous
import functools

import jax
import jax.numpy as jnp
from jax import lax
from jax.experimental import pallas as pl
from jax.experimental.pallas import tpu as pltpu
from jax.experimental.pallas import tpu_sc as plsc

F32 = jnp.float32
BF16 = jnp.bfloat16


def _bdot(a, b):
    return jnp.dot(a.astype(BF16), b.astype(BF16), preferred_element_type=F32)

_BN = 2048
_BE = 2048
_C = 2048
_NW = 32


def _cdiv(a, b):
    return (a + b - 1) // b


def _relu_mm_call(x, w, b, bn):
    n, k = x.shape
    m = w.shape[1]

    def body(x_ref, w_ref, b_ref, o_ref):
        acc = _bdot(x_ref[...], w_ref[...])
        o_ref[...] = jnp.maximum(acc + b_ref[...], 0.0)

    return pl.pallas_call(
        body,
        grid=(n // bn,),
        in_specs=[
            pl.BlockSpec((bn, k), lambda i: (i, 0)),
            pl.BlockSpec((k, m), lambda i: (0, 0)),
            pl.BlockSpec((1, m), lambda i: (0, 0)),
        ],
        out_specs=pl.BlockSpec((bn, m), lambda i: (i, 0)),
        out_shape=jax.ShapeDtypeStruct((n, m), F32),
    )(x, w, b)


def _msg_call(pa, xd, xs, w1, b1, w2, b2, expand):
    e, d = xd.shape
    h = w1.shape[1]
    dd2 = w2.shape[1]

    def body(pa_ref, xd_ref, xs_ref, w1_ref, b1_ref, w2_ref, b2_ref,
             exp_ref, o_ref):
        h1 = jnp.maximum(_bdot(pa_ref[...], w1_ref[...]) + b1_ref[...], 0.0)
        w = _bdot(h1, w2_ref[...]) + b2_ref[...]
        nf = jnp.concatenate([xd_ref[...], xs_ref[...]], axis=1)
        nfx = jnp.dot(nf, exp_ref[...], preferred_element_type=F32, precision=lax.Precision.HIGHEST)
        p = nfx * w
        width = dd2
        while width > d:
            width //= 2
            p = p[:, :width] + p[:, width:]
        o_ref[...] = p

    pdim = pa.shape[1]
    return pl.pallas_call(
        body,
        grid=(e // _BE,),
        in_specs=[
            pl.BlockSpec((_BE, pdim), lambda i: (i, 0)),
            pl.BlockSpec((_BE, d), lambda i: (i, 0)),
            pl.BlockSpec((_BE, d), lambda i: (i, 0)),
            pl.BlockSpec((pdim, h), lambda i: (0, 0)),
            pl.BlockSpec((1, h), lambda i: (0, 0)),
            pl.BlockSpec((h, dd2), lambda i: (0, 0)),
            pl.BlockSpec((1, dd2), lambda i: (0, 0)),
            pl.BlockSpec((2 * d, dd2), lambda i: (0, 0)),
        ],
        out_specs=pl.BlockSpec((_BE, d), lambda i: (i, 0)),
        out_shape=jax.ShapeDtypeStruct((e, d), F32),
    )(pa, xd, xs, w1, b1, w2, b2, expand)


def _combine_call(parts, hprev, root, bias):
    npad, d = hprev.shape

    def body(p_ref, h_ref, r_ref, b_ref, o_ref):
        t = p_ref[0] + p_ref[1]
        t = t + _bdot(h_ref[...], r_ref[...])
        o_ref[...] = t + b_ref[...]

    return pl.pallas_call(
        body,
        grid=(npad // _BN,),
        in_specs=[
            pl.BlockSpec((2, _BN, d), lambda i: (0, i, 0)),
            pl.BlockSpec((_BN, d), lambda i: (i, 0)),
            pl.BlockSpec((d, d), lambda i: (0, 0)),
            pl.BlockSpec((1, d), lambda i: (0, 0)),
        ],
        out_specs=pl.BlockSpec((_BN, d), lambda i: (i, 0)),
        out_shape=jax.ShapeDtypeStruct((npad, d), F32),
    )(parts, hprev, root, bias)


def _combine_final_call(parts, hprev, root, bias, wnt, bnt):
    npad, d = hprev.shape
    out_f = wnt.shape[1]

    def body(p_ref, h_ref, r_ref, b_ref, wnt_ref, bnt_ref, o_ref):
        t = p_ref[0] + p_ref[1]
        t = t + _bdot(h_ref[...], r_ref[...])
        t = t + b_ref[...]
        o = _bdot(t, wnt_ref[...]) + bnt_ref[...]
        o_ref[...] = jnp.maximum(o, 0.0)

    return pl.pallas_call(
        body,
        grid=(npad // _BN,),
        in_specs=[
            pl.BlockSpec((2, _BN, d), lambda i: (0, i, 0)),
            pl.BlockSpec((_BN, d), lambda i: (i, 0)),
            pl.BlockSpec((d, d), lambda i: (0, 0)),
            pl.BlockSpec((1, d), lambda i: (0, 0)),
            pl.BlockSpec((d, out_f), lambda i: (0, 0)),
            pl.BlockSpec((1, out_f), lambda i: (0, 0)),
        ],
        out_specs=pl.BlockSpec((_BN, out_f), lambda i: (i, 0)),
        out_shape=jax.ShapeDtypeStruct((npad, out_f), F32),
    )(parts, hprev, root, bias, wnt, bnt)


def _sc_gather(tbl, dst2, src2):
    npad, d = tbl.shape
    epad = dst2.shape[0] * 128
    ept = epad // _NW
    chunks = ept // _C
    mesh = plsc.VectorSubcoreMesh(core_axis_name="c", subcore_axis_name="s")

    @functools.partial(
        pl.kernel,
        out_type=[jax.ShapeDtypeStruct((epad, d), F32),
                  jax.ShapeDtypeStruct((epad, d), F32)],
        mesh=mesh,
        scratch_types=[
            pltpu.VMEM((_C // 128, 128), jnp.int32),
            pltpu.VMEM((_C // 128, 128), jnp.int32),
            pltpu.VMEM((_C, d), F32),
            pltpu.VMEM((_C, d), F32),
            pltpu.SemaphoreType.DMA,
            pltpu.SemaphoreType.DMA,
        ],
        compiler_params=pltpu.CompilerParams(use_tc_tiling_on_sc=False),
    )
    def k(tbl_hbm, dst_hbm, src_hbm, xd_hbm, xs_hbm,
          idx_d, idx_s, rows_d, rows_s, sem_d, sem_s):
        cid = lax.axis_index("c")
        sid = lax.axis_index("s")
        wid = cid * 16 + sid
        for j in range(chunks):
            row0 = wid * (ept // 128) + j * (_C // 128)
            pltpu.sync_copy(dst_hbm.at[pl.ds(row0, _C // 128)], idx_d)
            pltpu.sync_copy(src_hbm.at[pl.ds(row0, _C // 128)], idx_s)
            cps = []
            for t in range(_C // 128):
                cps.append(pltpu.async_copy(
                    tbl_hbm.at[idx_d.at[t]],
                    rows_d.at[pl.ds(t * 128, 128)], sem_d))
                cps.append(pltpu.async_copy(
                    tbl_hbm.at[idx_s.at[t]],
                    rows_s.at[pl.ds(t * 128, 128)], sem_s))
            for cp in cps:
                cp.wait()
            base = wid * ept + j * _C
            pltpu.sync_copy(rows_d, xd_hbm.at[pl.ds(base, _C)])
            pltpu.sync_copy(rows_s, xs_hbm.at[pl.ds(base, _C)])

    return k(tbl, dst2, src2)


def _sc_scatter(msg, dst2, ztbl):
    npad, d = ztbl.shape
    epad = dst2.shape[0] * 128
    ept = epad // _NW
    chunks = ept // _C
    stripe = npad // 16
    mesh = plsc.VectorSubcoreMesh(core_axis_name="c", subcore_axis_name="s")

    @functools.partial(
        pl.kernel,
        out_type=jax.ShapeDtypeStruct((2 * npad, d), F32),
        mesh=mesh,
        scratch_types=[
            pltpu.VMEM((_C // 128, 128), jnp.int32),
            pltpu.VMEM((_C, d), F32),
            pltpu.VMEM_SHARED((npad, d), F32),
        ],
        compiler_params=pltpu.CompilerParams(use_tc_tiling_on_sc=False),
    )
    def k(msg_hbm, dst_hbm, z_hbm, out_hbm, idx_v, rows_v, shared):
        cid = lax.axis_index("c")
        sid = lax.axis_index("s")
        pltpu.sync_copy(z_hbm.at[pl.ds(sid * stripe, stripe)],
                        shared.at[pl.ds(sid * stripe, stripe)])
        plsc.subcore_barrier()
        for j in range(chunks):
            base = (cid * 16 + sid) * ept + j * _C
            pltpu.sync_copy(dst_hbm.at[pl.ds(base // 128, _C // 128)], idx_v)
            pltpu.sync_copy(msg_hbm.at[pl.ds(base, _C)], rows_v)
            for t in range(_C // 128):
                pltpu.sync_copy(rows_v.at[pl.ds(t * 128, 128)],
                                shared.at[idx_v.at[t]], add=True)
        plsc.subcore_barrier()
        pltpu.sync_copy(shared.at[pl.ds(sid * stripe, stripe)],
                        out_hbm.at[pl.ds(cid * npad + sid * stripe, stripe)])

    return k(msg, dst2, ztbl)


def _conv(hcur, dst2, src2, pa, w1, b1, w2, b2, expand, root, bias, ztbl):
    xd, xs = _sc_gather(hcur, dst2, src2)
    msg = _msg_call(pa, xd, xs, w1, b1, w2, b2, expand)
    parts = _sc_scatter(msg, dst2, ztbl)
    npad, d = hcur.shape
    return _combine_call(parts.reshape(2, npad, d), hcur, root, bias)


def kernel(x, edge_index, edge_attr, W_ne, b_ne, W_ee, b_ee, We1, be1, We2,
           be2, root_e1, bias_e1, root_e2, bias_e2, Wd1, bd1, Wd2, bd2,
           root_d1, bias_d1, root_d2, bias_d2, W_nt, b_nt):
    n, node_f = x.shape
    e, edge_f = edge_attr.shape
    d = W_ne.shape[1]

    npad = _cdiv(n, _BN) * _BN
    epad = _cdiv(e, _NW * _C) * (_NW * _C)

    src = edge_index[0]
    dst = edge_index[1]
    src_p = jnp.concatenate([src, jnp.zeros((epad - e,), jnp.int32)])
    dst_p = jnp.concatenate(
        [dst, jnp.full((epad - e,), npad - 1, jnp.int32)])
    src2 = src_p.reshape(epad // 128, 128)
    dst2 = dst_p.reshape(epad // 128, 128)

    x_p = jnp.pad(x, ((0, npad - n), (0, 0)))
    ea_p = jnp.pad(edge_attr, ((0, epad - e), (0, 0)))
    ztbl = jnp.zeros((npad, d), F32)

    expand = jnp.repeat(jnp.eye(2 * d, dtype=F32), d, axis=1)

    r2 = lambda v: v.reshape(1, -1)

    h = _relu_mm_call(x_p, W_ne, r2(b_ne), _BN)
    ee = _relu_mm_call(ea_p, W_ee, r2(b_ee), 8192)

    h = _conv(h, dst2, src2, ea_p, We1, r2(be1), We2, r2(be2), expand,
              root_e1, r2(bias_e1), ztbl)
    h = _conv(h, dst2, src2, ea_p, We1, r2(be1), We2, r2(be2), expand,
              root_e2, r2(bias_e2), ztbl)
    h = _conv(h, dst2, src2, ee, Wd1, r2(bd1), Wd2, r2(bd2), expand,
              root_d1, r2(bias_d1), ztbl)

    xd, xs = _sc_gather(h, dst2, src2)
    msg = _msg_call(ee, xd, xs, Wd1, r2(bd1), Wd2, r2(bd2), expand)
    parts = _sc_scatter(msg, dst2, ztbl)
    out = _combine_final_call(parts.reshape(2, npad, d), h, root_d2,
                              r2(bias_d2), W_nt, r2(b_nt))
    return out[:n]


# speedup vs baseline: 4.5341x; 2.4138x over previous
import functools

import jax
import jax.numpy as jnp
from jax import lax
from jax.experimental import pallas as pl
from jax.experimental.pallas import tpu as pltpu
from jax.experimental.pallas import tpu_sc as plsc

F32 = jnp.float32

_BE = 2048
_C = 2048
_NW = 32


def _cdiv(a, b):
    return (a + b - 1) // b


def _kron8(m):
    return jnp.kron(jnp.eye(8, dtype=F32), m)


def _tile8(v):
    return jnp.tile(v.reshape(1, -1), (1, 8))


def _node_enc_call(xblk, wblk, bblk):
    rows = xblk.shape[0]
    kk = xblk.shape[1]

    def body(x_ref, w_ref, b_ref, o_ref):
        acc = jnp.dot(x_ref[...], w_ref[...], preferred_element_type=F32)
        o_ref[...] = jnp.maximum(acc + b_ref[...], 0.0)

    return pl.pallas_call(
        body,
        grid=(rows // 256,),
        in_specs=[
            pl.BlockSpec((256, kk), lambda i: (i, 0)),
            pl.BlockSpec((kk, 128), lambda i: (0, 0)),
            pl.BlockSpec((1, 128), lambda i: (0, 0)),
        ],
        out_specs=pl.BlockSpec((256, 128), lambda i: (i, 0)),
        out_shape=jax.ShapeDtypeStruct((rows, 128), F32),
    )(xblk, wblk, bblk)


def _edge_prep_call(ea128, we1b, be1b, weeb, beeb, wd1b, bd1b):
    rows = ea128.shape[0]

    def body(ea_ref, w1_ref, b1_ref, we_ref, be_ref, wd_ref, bd_ref,
             h1e_ref, ee_ref, h1d_ref):
        ea = ea_ref[...]
        h1e_ref[...] = jnp.maximum(
            jnp.dot(ea, w1_ref[...], preferred_element_type=F32)
            + b1_ref[...], 0.0)
        ee = jnp.maximum(
            jnp.dot(ea, we_ref[...], preferred_element_type=F32)
            + be_ref[...], 0.0)
        ee_ref[...] = ee
        h1d_ref[...] = jnp.maximum(
            jnp.dot(ee, wd_ref[...], preferred_element_type=F32)
            + bd_ref[...], 0.0)

    return pl.pallas_call(
        body,
        grid=(rows // 2048,),
        in_specs=[
            pl.BlockSpec((2048, 128), lambda i: (i, 0)),
            pl.BlockSpec((128, 256), lambda i: (0, 0)),
            pl.BlockSpec((1, 256), lambda i: (0, 0)),
            pl.BlockSpec((128, 128), lambda i: (0, 0)),
            pl.BlockSpec((1, 128), lambda i: (0, 0)),
            pl.BlockSpec((128, 256), lambda i: (0, 0)),
            pl.BlockSpec((1, 256), lambda i: (0, 0)),
        ],
        out_specs=[
            pl.BlockSpec((2048, 256), lambda i: (i, 0)),
            pl.BlockSpec((2048, 128), lambda i: (i, 0)),
            pl.BlockSpec((2048, 256), lambda i: (i, 0)),
        ],
        out_shape=[
            jax.ShapeDtypeStruct((rows, 256), F32),
            jax.ShapeDtypeStruct((rows, 128), F32),
            jax.ShapeDtypeStruct((rows, 256), F32),
        ],
    )(ea128, we1b, be1b, weeb, beeb, wd1b, bd1b)


def _msg_call(h1p, xd128, xs128, w2blk, b2blk, expblk):
    rows = h1p.shape[0]

    def body(h1_ref, xd_ref, xs_ref, w2_ref, b2_ref, exp_ref, o_ref):
        wp = jnp.dot(h1_ref[...], w2_ref[...],
                     preferred_element_type=F32) + b2_ref[...]
        nf = jnp.concatenate([xd_ref[...], xs_ref[...]], axis=1)
        nf_hi = nf.astype(jnp.bfloat16)
        nf_lo = (nf - nf_hi.astype(F32)).astype(jnp.bfloat16)
        ex = exp_ref[...]
        nfx = (jnp.dot(nf_hi, ex, preferred_element_type=F32)
               + jnp.dot(nf_lo, ex, preferred_element_type=F32))
        p = nfx * wp
        w = 4096
        while w > 128:
            w //= 2
            p = p[:, :w] + p[:, w:]
        o_ref[...] = p

    return pl.pallas_call(
        body,
        grid=(rows // 256,),
        in_specs=[
            pl.BlockSpec((256, 256), lambda i: (i, 0)),
            pl.BlockSpec((256, 128), lambda i: (i, 0)),
            pl.BlockSpec((256, 128), lambda i: (i, 0)),
            pl.BlockSpec((256, 4096), lambda i: (0, 0)),
            pl.BlockSpec((1, 4096), lambda i: (0, 0)),
            pl.BlockSpec((256, 4096), lambda i: (0, 0)),
        ],
        out_specs=pl.BlockSpec((256, 128), lambda i: (i, 0)),
        out_shape=jax.ShapeDtypeStruct((rows, 128), F32),
    )(h1p, xd128, xs128, w2blk, b2blk, expblk)


def _combine_call(parts, hp, rootb, biasb):
    prow = hp.shape[0]
    nb = prow // 256

    def body(p0_ref, p1_ref, h_ref, r_ref, b_ref, o_ref):
        t = p0_ref[...] + p1_ref[...]
        t = t + jnp.dot(h_ref[...], r_ref[...], preferred_element_type=F32)
        o_ref[...] = t + b_ref[...]

    return pl.pallas_call(
        body,
        grid=(nb,),
        in_specs=[
            pl.BlockSpec((256, 128), lambda i: (i, 0)),
            pl.BlockSpec((256, 128), lambda i, _nb=nb: (_nb + i, 0)),
            pl.BlockSpec((256, 128), lambda i: (i, 0)),
            pl.BlockSpec((128, 128), lambda i: (0, 0)),
            pl.BlockSpec((1, 128), lambda i: (0, 0)),
        ],
        out_specs=pl.BlockSpec((256, 128), lambda i: (i, 0)),
        out_shape=jax.ShapeDtypeStruct((prow, 128), F32),
    )(parts, parts, hp, rootb, biasb)


def _combine_final_call(parts, hp, rootb, biasb, outb, bntb):
    prow = hp.shape[0]
    nb = prow // 256

    def body(p0_ref, p1_ref, h_ref, r_ref, b_ref, w_ref, bn_ref, o_ref):
        t = p0_ref[...] + p1_ref[...]
        t = t + jnp.dot(h_ref[...], r_ref[...], preferred_element_type=F32)
        t = t + b_ref[...]
        o = jnp.dot(t, w_ref[...], preferred_element_type=F32) + bn_ref[...]
        o_ref[...] = jnp.maximum(o, 0.0)

    return pl.pallas_call(
        body,
        grid=(nb,),
        in_specs=[
            pl.BlockSpec((256, 128), lambda i: (i, 0)),
            pl.BlockSpec((256, 128), lambda i, _nb=nb: (_nb + i, 0)),
            pl.BlockSpec((256, 128), lambda i: (i, 0)),
            pl.BlockSpec((128, 128), lambda i: (0, 0)),
            pl.BlockSpec((1, 128), lambda i: (0, 0)),
            pl.BlockSpec((128, 16), lambda i: (0, 0)),
            pl.BlockSpec((1, 16), lambda i: (0, 0)),
        ],
        out_specs=pl.BlockSpec((256, 16), lambda i: (i, 0)),
        out_shape=jax.ShapeDtypeStruct((prow, 16), F32),
    )(parts, parts, hp, rootb, biasb, outb, bntb)


def _sc_gather(tbln, dst2, src2):
    npad = tbln.shape[0]
    epad = dst2.shape[0] * 128
    ept = epad // _NW
    chunks = ept // _C
    mesh = plsc.VectorSubcoreMesh(core_axis_name="c", subcore_axis_name="s")

    cg = 1024
    nch = ept // cg
    rows_per = ept // 128

    @functools.partial(
        pl.kernel,
        out_type=[jax.ShapeDtypeStruct((epad, 16), F32),
                  jax.ShapeDtypeStruct((epad, 16), F32)],
        mesh=mesh,
        scratch_types=[
            pltpu.VMEM((rows_per, 128), jnp.int32),
            pltpu.VMEM((rows_per, 128), jnp.int32),
            pltpu.VMEM((2, cg, 16), F32),
            pltpu.VMEM((2, cg, 16), F32),
            pltpu.SemaphoreType.DMA,
            pltpu.SemaphoreType.DMA,
            pltpu.SemaphoreType.DMA,
            pltpu.SemaphoreType.DMA,
        ],
        compiler_params=pltpu.CompilerParams(use_tc_tiling_on_sc=False),
    )
    def k(tbl_hbm, dst_hbm, src_hbm, xd_hbm, xs_hbm,
          idx_d, idx_s, rows_d, rows_s, sem_g, sem_g2, sem_w0, sem_w1):
        cid = lax.axis_index("c")
        sid = lax.axis_index("s")
        wid = cid * 16 + sid
        pltpu.sync_copy(dst_hbm.at[pl.ds(wid * rows_per, rows_per)], idx_d)
        pltpu.sync_copy(src_hbm.at[pl.ds(wid * rows_per, rows_per)], idx_s)
        sem_gs = (sem_g, sem_g2)
        sem_w = (sem_w0, sem_w1)
        pend_g = [None, None]
        pend_w = [None, None]

        def fire(j):
            slot = j % 2
            if pend_w[slot] is not None:
                for cp in pend_w[slot]:
                    cp.wait()
                pend_w[slot] = None
            gcps = []
            for t in range(cg // 128):
                r = j * (cg // 128) + t
                gcps.append(pltpu.async_copy(
                    tbl_hbm.at[idx_d.at[r]],
                    rows_d.at[slot, pl.ds(t * 128, 128)], sem_gs[slot]))
                gcps.append(pltpu.async_copy(
                    tbl_hbm.at[idx_s.at[r]],
                    rows_s.at[slot, pl.ds(t * 128, 128)], sem_gs[slot]))
            pend_g[slot] = gcps

        fire(0)
        for j in range(nch):
            slot = j % 2
            if j + 1 < nch:
                fire(j + 1)
            for cp in pend_g[slot]:
                cp.wait()
            base = wid * ept + j * cg
            pend_w[slot] = [
                pltpu.async_copy(rows_d.at[slot],
                                 xd_hbm.at[pl.ds(base, cg)], sem_w[slot]),
                pltpu.async_copy(rows_s.at[slot],
                                 xs_hbm.at[pl.ds(base, cg)], sem_w[slot]),
            ]
        for slot in range(2):
            if pend_w[slot] is not None:
                for cp in pend_w[slot]:
                    cp.wait()

    return k(tbln, dst2, src2)


def _sc_scatter(msg128, dst2, ztbl):
    npad = ztbl.shape[0]
    epad = dst2.shape[0] * 128
    ept = epad // _NW
    chunks = ept // _C
    stripe = npad // 16
    mesh = plsc.VectorSubcoreMesh(core_axis_name="c", subcore_axis_name="s")

    @functools.partial(
        pl.kernel,
        out_type=jax.ShapeDtypeStruct((2 * npad, 16), F32),
        mesh=mesh,
        scratch_types=[
            pltpu.VMEM((_C // 128, 128), jnp.int32),
            pltpu.VMEM((_C, 16), F32),
            pltpu.VMEM_SHARED((npad, 16), F32),
        ],
        compiler_params=pltpu.CompilerParams(use_tc_tiling_on_sc=False),
    )
    def k(msg_hbm, dst_hbm, z_hbm, out_hbm, idx_v, rows_v, shared):
        cid = lax.axis_index("c")
        sid = lax.axis_index("s")
        pltpu.sync_copy(z_hbm.at[pl.ds(sid * stripe, stripe)],
                        shared.at[pl.ds(sid * stripe, stripe)])
        plsc.subcore_barrier()
        for j in range(chunks):
            base = (cid * 16 + sid) * ept + j * _C
            pltpu.sync_copy(dst_hbm.at[pl.ds(base // 128, _C // 128)], idx_v)
            pltpu.sync_copy(msg_hbm.at[pl.ds(base, _C)], rows_v)
            for t in range(_C // 128):
                pltpu.sync_copy(rows_v.at[pl.ds(t * 128, 128)],
                                shared.at[idx_v.at[t]], add=True)
        plsc.subcore_barrier()
        pltpu.sync_copy(shared.at[pl.ds(sid * stripe, stripe)],
                        out_hbm.at[pl.ds(cid * npad + sid * stripe, stripe)])

    return k(msg128, dst2, ztbl)


def _conv(hp, dst2, src2, h1p, w2blk, b2blk, expblk, ztbl):
    npad = hp.shape[0] * 8
    epad = dst2.shape[0] * 128
    xd, xs = _sc_gather(hp.reshape(npad, 16), dst2, src2)
    msg128 = _msg_call(h1p, xd.reshape(epad // 8, 128),
                       xs.reshape(epad // 8, 128), w2blk, b2blk, expblk)
    parts = _sc_scatter(msg128.reshape(epad, 16), dst2, ztbl)
    return parts.reshape(npad // 4, 128)


def kernel(x, edge_index, edge_attr, W_ne, b_ne, W_ee, b_ee, We1, be1, We2,
           be2, root_e1, bias_e1, root_e2, bias_e2, Wd1, bd1, Wd2, bd2,
           root_d1, bias_d1, root_d2, bias_d2, W_nt, b_nt):
    n, node_f = x.shape
    e, edge_f = edge_attr.shape

    npad = _cdiv(n, 2048) * 2048
    epad = _cdiv(e, _NW * _C) * (_NW * _C)

    src = edge_index[0]
    dst = edge_index[1]
    src_p = jnp.concatenate([src, jnp.zeros((epad - e,), jnp.int32)])
    dst_p = jnp.concatenate(
        [dst, jnp.full((epad - e,), npad - 1, jnp.int32)])
    src2 = src_p.reshape(epad // 128, 128)
    dst2 = dst_p.reshape(epad // 128, 128)

    xblk = jnp.pad(x, ((0, npad - n), (0, 0))).reshape(npad // 8, 8 * node_f)
    ea128 = jnp.pad(edge_attr.reshape(e // 8, 128),
                    ((0, (epad - e) // 8), (0, 0)))
    ztbl = jnp.zeros((npad, 16), F32)

    j = jnp.arange(256)
    e_j = jnp.where(j < 128, j // 16, (j - 128) // 16)
    c_j = jnp.where(j < 128, j % 16, 16 + (j - 128) % 16)
    col = jnp.arange(4096)
    expblk = ((col[None, :] // 128 == c_j[:, None])
              & ((col[None, :] % 128) // 16 == e_j[:, None])).astype(jnp.bfloat16)

    def _w2com(w2):
        return (_kron8(w2).reshape(256, 8, 32, 16)
                .transpose(0, 2, 1, 3).reshape(256, 4096))

    def _b2com(b2):
        return jnp.broadcast_to(b2.reshape(32, 1, 16),
                                (32, 8, 16)).reshape(1, 4096)

    w2eb, b2eb = _w2com(We2), _b2com(be2)
    w2db, b2db = _w2com(Wd2), _b2com(bd2)

    h0p = _node_enc_call(xblk, _kron8(W_ne), _tile8(b_ne))
    h1ep, eep, h1dp = _edge_prep_call(
        ea128, _kron8(We1), _tile8(be1), _kron8(W_ee), _tile8(b_ee),
        _kron8(Wd1), _tile8(bd1))

    parts = _conv(h0p, dst2, src2, h1ep, w2eb, b2eb, expblk, ztbl)
    hp = _combine_call(parts, h0p, _kron8(root_e1), _tile8(bias_e1))
    parts = _conv(hp, dst2, src2, h1ep, w2eb, b2eb, expblk, ztbl)
    hp = _combine_call(parts, hp, _kron8(root_e2), _tile8(bias_e2))
    parts = _conv(hp, dst2, src2, h1dp, w2db, b2db, expblk, ztbl)
    hp = _combine_call(parts, hp, _kron8(root_d1), _tile8(bias_d1))
    parts = _conv(hp, dst2, src2, h1dp, w2db, b2db, expblk, ztbl)
    outp = _combine_final_call(parts, hp, _kron8(root_d2), _tile8(bias_d2),
                               _kron8(W_nt), _tile8(b_nt))
    return outp.reshape(npad, 2)[:n]


# speedup vs baseline: 4.5352x; 1.0003x over previous
import functools

import jax
import jax.numpy as jnp
from jax import lax
from jax.experimental import pallas as pl
from jax.experimental.pallas import tpu as pltpu
from jax.experimental.pallas import tpu_sc as plsc

F32 = jnp.float32

_BE = 2048
_C = 2048
_NW = 32


def _cdiv(a, b):
    return (a + b - 1) // b


def _kron8(m):
    return jnp.kron(jnp.eye(8, dtype=F32), m)


def _tile8(v):
    return jnp.tile(v.reshape(1, -1), (1, 8))


def _node_enc_call(xblk, wblk, bblk):
    rows = xblk.shape[0]
    kk = xblk.shape[1]

    def body(x_ref, w_ref, b_ref, o_ref):
        acc = jnp.dot(x_ref[...], w_ref[...], preferred_element_type=F32)
        o_ref[...] = jnp.maximum(acc + b_ref[...], 0.0)

    return pl.pallas_call(
        body,
        grid=(rows // 256,),
        in_specs=[
            pl.BlockSpec((256, kk), lambda i: (i, 0)),
            pl.BlockSpec((kk, 128), lambda i: (0, 0)),
            pl.BlockSpec((1, 128), lambda i: (0, 0)),
        ],
        out_specs=pl.BlockSpec((256, 128), lambda i: (i, 0)),
        out_shape=jax.ShapeDtypeStruct((rows, 128), F32),
    )(xblk, wblk, bblk)


def _edge_prep_call(ea128, we1b, be1b, weeb, beeb, wd1b, bd1b):
    rows = ea128.shape[0]

    def body(ea_ref, w1_ref, b1_ref, we_ref, be_ref, wd_ref, bd_ref,
             h1e_ref, ee_ref, h1d_ref):
        ea = ea_ref[...]
        h1e_ref[...] = jnp.maximum(
            jnp.dot(ea, w1_ref[...], preferred_element_type=F32)
            + b1_ref[...], 0.0)
        ee = jnp.maximum(
            jnp.dot(ea, we_ref[...], preferred_element_type=F32)
            + be_ref[...], 0.0)
        ee_ref[...] = ee
        h1d_ref[...] = jnp.maximum(
            jnp.dot(ee, wd_ref[...], preferred_element_type=F32)
            + bd_ref[...], 0.0)

    return pl.pallas_call(
        body,
        grid=(rows // 2048,),
        in_specs=[
            pl.BlockSpec((2048, 128), lambda i: (i, 0)),
            pl.BlockSpec((128, 256), lambda i: (0, 0)),
            pl.BlockSpec((1, 256), lambda i: (0, 0)),
            pl.BlockSpec((128, 128), lambda i: (0, 0)),
            pl.BlockSpec((1, 128), lambda i: (0, 0)),
            pl.BlockSpec((128, 256), lambda i: (0, 0)),
            pl.BlockSpec((1, 256), lambda i: (0, 0)),
        ],
        out_specs=[
            pl.BlockSpec((2048, 256), lambda i: (i, 0)),
            pl.BlockSpec((2048, 128), lambda i: (i, 0)),
            pl.BlockSpec((2048, 256), lambda i: (i, 0)),
        ],
        out_shape=[
            jax.ShapeDtypeStruct((rows, 256), F32),
            jax.ShapeDtypeStruct((rows, 128), F32),
            jax.ShapeDtypeStruct((rows, 256), F32),
        ],
    )(ea128, we1b, be1b, weeb, beeb, wd1b, bd1b)


def _msg_call(h1p, xd128, xs128, w2blk, b2blk, expblk):
    rows = h1p.shape[0]

    def body(h1_ref, xd_ref, xs_ref, w2_ref, b2_ref, exp_ref, o_ref):
        wp = jnp.dot(h1_ref[...], w2_ref[...],
                     preferred_element_type=F32) + b2_ref[...]
        nf = jnp.concatenate([xd_ref[...], xs_ref[...]], axis=1)
        nf_hi = nf.astype(jnp.bfloat16)
        nf_lo = (nf - nf_hi.astype(F32)).astype(jnp.bfloat16)
        ex = exp_ref[...]
        nfx = (jnp.dot(nf_hi, ex, preferred_element_type=F32)
               + jnp.dot(nf_lo, ex, preferred_element_type=F32))
        p = nfx * wp
        w = 4096
        while w > 128:
            w //= 2
            p = p[:, :w] + p[:, w:]
        o_ref[...] = p

    return pl.pallas_call(
        body,
        grid=(rows // 256,),
        in_specs=[
            pl.BlockSpec((256, 256), lambda i: (i, 0)),
            pl.BlockSpec((256, 128), lambda i: (i, 0)),
            pl.BlockSpec((256, 128), lambda i: (i, 0)),
            pl.BlockSpec((256, 4096), lambda i: (0, 0)),
            pl.BlockSpec((1, 4096), lambda i: (0, 0)),
            pl.BlockSpec((256, 4096), lambda i: (0, 0)),
        ],
        out_specs=pl.BlockSpec((256, 128), lambda i: (i, 0)),
        out_shape=jax.ShapeDtypeStruct((rows, 128), F32),
    )(h1p, xd128, xs128, w2blk, b2blk, expblk)


def _combine_call(parts, hp, rootb, biasb):
    prow = hp.shape[0]
    nb = prow // 256

    def body(p0_ref, p1_ref, h_ref, r_ref, b_ref, o_ref):
        t = p0_ref[...] + p1_ref[...]
        t = t + jnp.dot(h_ref[...], r_ref[...], preferred_element_type=F32)
        o_ref[...] = t + b_ref[...]

    return pl.pallas_call(
        body,
        grid=(nb,),
        in_specs=[
            pl.BlockSpec((256, 128), lambda i: (i, 0)),
            pl.BlockSpec((256, 128), lambda i, _nb=nb: (_nb + i, 0)),
            pl.BlockSpec((256, 128), lambda i: (i, 0)),
            pl.BlockSpec((128, 128), lambda i: (0, 0)),
            pl.BlockSpec((1, 128), lambda i: (0, 0)),
        ],
        out_specs=pl.BlockSpec((256, 128), lambda i: (i, 0)),
        out_shape=jax.ShapeDtypeStruct((prow, 128), F32),
    )(parts, parts, hp, rootb, biasb)


def _combine_final_call(parts, hp, rootb, biasb, outb, bntb):
    prow = hp.shape[0]
    nb = prow // 256

    def body(p0_ref, p1_ref, h_ref, r_ref, b_ref, w_ref, bn_ref, o_ref):
        t = p0_ref[...] + p1_ref[...]
        t = t + jnp.dot(h_ref[...], r_ref[...], preferred_element_type=F32)
        t = t + b_ref[...]
        o = jnp.dot(t, w_ref[...], preferred_element_type=F32) + bn_ref[...]
        o_ref[...] = jnp.maximum(o, 0.0)

    return pl.pallas_call(
        body,
        grid=(nb,),
        in_specs=[
            pl.BlockSpec((256, 128), lambda i: (i, 0)),
            pl.BlockSpec((256, 128), lambda i, _nb=nb: (_nb + i, 0)),
            pl.BlockSpec((256, 128), lambda i: (i, 0)),
            pl.BlockSpec((128, 128), lambda i: (0, 0)),
            pl.BlockSpec((1, 128), lambda i: (0, 0)),
            pl.BlockSpec((128, 16), lambda i: (0, 0)),
            pl.BlockSpec((1, 16), lambda i: (0, 0)),
        ],
        out_specs=pl.BlockSpec((256, 16), lambda i: (i, 0)),
        out_shape=jax.ShapeDtypeStruct((prow, 16), F32),
    )(parts, parts, hp, rootb, biasb, outb, bntb)


def _sc_gather(tbln, dst2, src2):
    npad = tbln.shape[0]
    epad = dst2.shape[0] * 128
    ept = epad // _NW
    chunks = ept // _C
    mesh = plsc.VectorSubcoreMesh(core_axis_name="c", subcore_axis_name="s")

    cg = 1024
    nch = ept // cg
    rows_per = ept // 128

    @functools.partial(
        pl.kernel,
        out_type=[jax.ShapeDtypeStruct((epad, 16), F32),
                  jax.ShapeDtypeStruct((epad, 16), F32)],
        mesh=mesh,
        scratch_types=[
            pltpu.VMEM((rows_per, 128), jnp.int32),
            pltpu.VMEM((rows_per, 128), jnp.int32),
            pltpu.VMEM((2, cg, 16), F32),
            pltpu.VMEM((2, cg, 16), F32),
            pltpu.SemaphoreType.DMA,
            pltpu.SemaphoreType.DMA,
            pltpu.SemaphoreType.DMA,
        ],
        compiler_params=pltpu.CompilerParams(use_tc_tiling_on_sc=False),
    )
    def k(tbl_hbm, dst_hbm, src_hbm, xd_hbm, xs_hbm,
          idx_d, idx_s, rows_d, rows_s, sem_g, sem_w0, sem_w1):
        cid = lax.axis_index("c")
        sid = lax.axis_index("s")
        wid = cid * 16 + sid
        pltpu.sync_copy(dst_hbm.at[pl.ds(wid * rows_per, rows_per)], idx_d)
        pltpu.sync_copy(src_hbm.at[pl.ds(wid * rows_per, rows_per)], idx_s)
        sem_w = (sem_w0, sem_w1)
        pend = [None, None]
        for j in range(nch):
            slot = j % 2
            if pend[slot] is not None:
                for cp in pend[slot]:
                    cp.wait()
            gcps = []
            for t in range(cg // 128):
                r = j * (cg // 128) + t
                gcps.append(pltpu.async_copy(
                    tbl_hbm.at[idx_d.at[r]],
                    rows_d.at[slot, pl.ds(t * 128, 128)], sem_g))
                gcps.append(pltpu.async_copy(
                    tbl_hbm.at[idx_s.at[r]],
                    rows_s.at[slot, pl.ds(t * 128, 128)], sem_g))
            for cp in gcps:
                cp.wait()
            base = wid * ept + j * cg
            pend[slot] = [
                pltpu.async_copy(rows_d.at[slot],
                                 xd_hbm.at[pl.ds(base, cg)], sem_w[slot]),
                pltpu.async_copy(rows_s.at[slot],
                                 xs_hbm.at[pl.ds(base, cg)], sem_w[slot]),
            ]
        for slot in range(2):
            if pend[slot] is not None:
                for cp in pend[slot]:
                    cp.wait()

    return k(tbln, dst2, src2)


def _sc_scatter(msg128, dst2, ztbl):
    npad = ztbl.shape[0]
    epad = dst2.shape[0] * 128
    ept = epad // _NW
    chunks = ept // _C
    stripe = npad // 16
    mesh = plsc.VectorSubcoreMesh(core_axis_name="c", subcore_axis_name="s")

    @functools.partial(
        pl.kernel,
        out_type=jax.ShapeDtypeStruct((2 * npad, 16), F32),
        mesh=mesh,
        scratch_types=[
            pltpu.VMEM((_C // 128, 128), jnp.int32),
            pltpu.VMEM((_C, 16), F32),
            pltpu.VMEM_SHARED((npad, 16), F32),
        ],
        compiler_params=pltpu.CompilerParams(use_tc_tiling_on_sc=False),
    )
    def k(msg_hbm, dst_hbm, z_hbm, out_hbm, idx_v, rows_v, shared):
        cid = lax.axis_index("c")
        sid = lax.axis_index("s")
        pltpu.sync_copy(z_hbm.at[pl.ds(sid * stripe, stripe)],
                        shared.at[pl.ds(sid * stripe, stripe)])
        plsc.subcore_barrier()
        for j in range(chunks):
            base = (cid * 16 + sid) * ept + j * _C
            pltpu.sync_copy(dst_hbm.at[pl.ds(base // 128, _C // 128)], idx_v)
            pltpu.sync_copy(msg_hbm.at[pl.ds(base, _C)], rows_v)
            for t in range(_C // 128):
                pltpu.sync_copy(rows_v.at[pl.ds(t * 128, 128)],
                                shared.at[idx_v.at[t]], add=True)
        plsc.subcore_barrier()
        pltpu.sync_copy(shared.at[pl.ds(sid * stripe, stripe)],
                        out_hbm.at[pl.ds(cid * npad + sid * stripe, stripe)])

    return k(msg128, dst2, ztbl)


def _conv(hp, dst2, src2, h1p, w2blk, b2blk, expblk, ztbl):
    npad = hp.shape[0] * 8
    epad = dst2.shape[0] * 128
    xd, xs = _sc_gather(hp.reshape(npad, 16), dst2, src2)
    msg128 = _msg_call(h1p, xd.reshape(epad // 8, 128),
                       xs.reshape(epad // 8, 128), w2blk, b2blk, expblk)
    parts = _sc_scatter(msg128.reshape(epad, 16), dst2, ztbl)
    return parts.reshape(npad // 4, 128)


def kernel(x, edge_index, edge_attr, W_ne, b_ne, W_ee, b_ee, We1, be1, We2,
           be2, root_e1, bias_e1, root_e2, bias_e2, Wd1, bd1, Wd2, bd2,
           root_d1, bias_d1, root_d2, bias_d2, W_nt, b_nt):
    n, node_f = x.shape
    e, edge_f = edge_attr.shape

    npad = _cdiv(n, 2048) * 2048
    epad = _cdiv(e, _NW * _C) * (_NW * _C)

    src = edge_index[0]
    dst = edge_index[1]
    src_p = jnp.concatenate([src, jnp.zeros((epad - e,), jnp.int32)])
    dst_p = jnp.concatenate(
        [dst, jnp.full((epad - e,), npad - 1, jnp.int32)])
    src2 = src_p.reshape(epad // 128, 128)
    dst2 = dst_p.reshape(epad // 128, 128)

    xblk = jnp.pad(x, ((0, npad - n), (0, 0))).reshape(npad // 8, 8 * node_f)
    ea128 = jnp.pad(edge_attr.reshape(e // 8, 128),
                    ((0, (epad - e) // 8), (0, 0)))
    ztbl = jnp.zeros((npad, 16), F32)

    j = jnp.arange(256)
    e_j = jnp.where(j < 128, j // 16, (j - 128) // 16)
    c_j = jnp.where(j < 128, j % 16, 16 + (j - 128) % 16)
    col = jnp.arange(4096)
    expblk = ((col[None, :] // 128 == c_j[:, None])
              & ((col[None, :] % 128) // 16 == e_j[:, None])).astype(jnp.bfloat16)

    def _w2com(w2):
        return (_kron8(w2).reshape(256, 8, 32, 16)
                .transpose(0, 2, 1, 3).reshape(256, 4096))

    def _b2com(b2):
        return jnp.broadcast_to(b2.reshape(32, 1, 16),
                                (32, 8, 16)).reshape(1, 4096)

    w2eb, b2eb = _w2com(We2), _b2com(be2)
    w2db, b2db = _w2com(Wd2), _b2com(bd2)

    h0p = _node_enc_call(xblk, _kron8(W_ne), _tile8(b_ne))
    h1ep, eep, h1dp = _edge_prep_call(
        ea128, _kron8(We1), _tile8(be1), _kron8(W_ee), _tile8(b_ee),
        _kron8(Wd1), _tile8(bd1))

    parts = _conv(h0p, dst2, src2, h1ep, w2eb, b2eb, expblk, ztbl)
    hp = _combine_call(parts, h0p, _kron8(root_e1), _tile8(bias_e1))
    parts = _conv(hp, dst2, src2, h1ep, w2eb, b2eb, expblk, ztbl)
    hp = _combine_call(parts, hp, _kron8(root_e2), _tile8(bias_e2))
    parts = _conv(hp, dst2, src2, h1dp, w2db, b2db, expblk, ztbl)
    hp = _combine_call(parts, hp, _kron8(root_d1), _tile8(bias_d1))
    parts = _conv(hp, dst2, src2, h1dp, w2db, b2db, expblk, ztbl)
    outp = _combine_final_call(parts, hp, _kron8(root_d2), _tile8(bias_d2),
                               _kron8(W_nt), _tile8(b_nt))
    return outp.reshape(npad, 2)[:n]


# speedup vs baseline: 4.5855x; 1.0111x over previous
import functools

import jax
import jax.numpy as jnp
from jax import lax
from jax.experimental import pallas as pl
from jax.experimental.pallas import tpu as pltpu
from jax.experimental.pallas import tpu_sc as plsc

F32 = jnp.float32

_BE = 2048
_C = 2048
_NW = 32


def _cdiv(a, b):
    return (a + b - 1) // b


def _kron8(m):
    return jnp.kron(jnp.eye(8, dtype=F32), m)


def _tile8(v):
    return jnp.tile(v.reshape(1, -1), (1, 8))


def _node_enc_call(xblk, wblk, bblk):
    rows = xblk.shape[0]
    kk = xblk.shape[1]

    def body(x_ref, w_ref, b_ref, o_ref):
        acc = jnp.dot(x_ref[...], w_ref[...], preferred_element_type=F32)
        o_ref[...] = jnp.maximum(acc + b_ref[...], 0.0)

    return pl.pallas_call(
        body,
        grid=(rows // 256,),
        in_specs=[
            pl.BlockSpec((256, kk), lambda i: (i, 0)),
            pl.BlockSpec((kk, 128), lambda i: (0, 0)),
            pl.BlockSpec((1, 128), lambda i: (0, 0)),
        ],
        out_specs=pl.BlockSpec((256, 128), lambda i: (i, 0)),
        out_shape=jax.ShapeDtypeStruct((rows, 128), F32),
    )(xblk, wblk, bblk)


def _edge_prep_call(ea128, we1b, be1b, weeb, beeb, wd1b, bd1b):
    rows = ea128.shape[0]

    def body(ea_ref, w1_ref, b1_ref, we_ref, be_ref, wd_ref, bd_ref,
             h1e_ref, ee_ref, h1d_ref):
        ea = ea_ref[...]
        h1e_ref[...] = jnp.maximum(
            jnp.dot(ea, w1_ref[...], preferred_element_type=F32)
            + b1_ref[...], 0.0)
        ee = jnp.maximum(
            jnp.dot(ea, we_ref[...], preferred_element_type=F32)
            + be_ref[...], 0.0)
        ee_ref[...] = ee
        h1d_ref[...] = jnp.maximum(
            jnp.dot(ee, wd_ref[...], preferred_element_type=F32)
            + bd_ref[...], 0.0)

    return pl.pallas_call(
        body,
        grid=(rows // 2048,),
        in_specs=[
            pl.BlockSpec((2048, 128), lambda i: (i, 0)),
            pl.BlockSpec((128, 256), lambda i: (0, 0)),
            pl.BlockSpec((1, 256), lambda i: (0, 0)),
            pl.BlockSpec((128, 128), lambda i: (0, 0)),
            pl.BlockSpec((1, 128), lambda i: (0, 0)),
            pl.BlockSpec((128, 256), lambda i: (0, 0)),
            pl.BlockSpec((1, 256), lambda i: (0, 0)),
        ],
        out_specs=[
            pl.BlockSpec((2048, 256), lambda i: (i, 0)),
            pl.BlockSpec((2048, 128), lambda i: (i, 0)),
            pl.BlockSpec((2048, 256), lambda i: (i, 0)),
        ],
        out_shape=[
            jax.ShapeDtypeStruct((rows, 256), F32),
            jax.ShapeDtypeStruct((rows, 128), F32),
            jax.ShapeDtypeStruct((rows, 256), F32),
        ],
    )(ea128, we1b, be1b, weeb, beeb, wd1b, bd1b)


def _msg_call(h1p, xd128, xs128, w2blk, b2blk, expblk):
    rows = h1p.shape[0]

    def body(h1_ref, xd_ref, xs_ref, w2_ref, b2_ref, exp_ref, o_ref):
        wp = jnp.dot(h1_ref[...], w2_ref[...],
                     preferred_element_type=F32) + b2_ref[...]
        nf = jnp.concatenate([xd_ref[...], xs_ref[...]], axis=1)
        nf_hi = nf.astype(jnp.bfloat16)
        nf_lo = (nf - nf_hi.astype(F32)).astype(jnp.bfloat16)
        ex = exp_ref[...]
        nfx = (jnp.dot(nf_hi, ex, preferred_element_type=F32)
               + jnp.dot(nf_lo, ex, preferred_element_type=F32))
        p = nfx * wp
        w = 4096
        while w > 128:
            w //= 2
            p = p[:, :w] + p[:, w:]
        o_ref[...] = p

    br = 512
    return pl.pallas_call(
        body,
        grid=(rows // br,),
        in_specs=[
            pl.BlockSpec((br, 256), lambda i: (i, 0)),
            pl.BlockSpec((br, 128), lambda i: (i, 0)),
            pl.BlockSpec((br, 128), lambda i: (i, 0)),
            pl.BlockSpec((256, 4096), lambda i: (0, 0)),
            pl.BlockSpec((1, 4096), lambda i: (0, 0)),
            pl.BlockSpec((256, 4096), lambda i: (0, 0)),
        ],
        out_specs=pl.BlockSpec((br, 128), lambda i: (i, 0)),
        out_shape=jax.ShapeDtypeStruct((rows, 128), F32),
    )(h1p, xd128, xs128, w2blk, b2blk, expblk)


def _combine_call(parts, hp, rootb, biasb):
    prow = hp.shape[0]
    nb = prow // 256

    def body(p0_ref, p1_ref, h_ref, r_ref, b_ref, o_ref):
        t = p0_ref[...] + p1_ref[...]
        t = t + jnp.dot(h_ref[...], r_ref[...], preferred_element_type=F32)
        o_ref[...] = t + b_ref[...]

    return pl.pallas_call(
        body,
        grid=(nb,),
        in_specs=[
            pl.BlockSpec((256, 128), lambda i: (i, 0)),
            pl.BlockSpec((256, 128), lambda i, _nb=nb: (_nb + i, 0)),
            pl.BlockSpec((256, 128), lambda i: (i, 0)),
            pl.BlockSpec((128, 128), lambda i: (0, 0)),
            pl.BlockSpec((1, 128), lambda i: (0, 0)),
        ],
        out_specs=pl.BlockSpec((256, 128), lambda i: (i, 0)),
        out_shape=jax.ShapeDtypeStruct((prow, 128), F32),
    )(parts, parts, hp, rootb, biasb)


def _combine_final_call(parts, hp, rootb, biasb, outb, bntb):
    prow = hp.shape[0]
    nb = prow // 256

    def body(p0_ref, p1_ref, h_ref, r_ref, b_ref, w_ref, bn_ref, o_ref):
        t = p0_ref[...] + p1_ref[...]
        t = t + jnp.dot(h_ref[...], r_ref[...], preferred_element_type=F32)
        t = t + b_ref[...]
        o = jnp.dot(t, w_ref[...], preferred_element_type=F32) + bn_ref[...]
        o_ref[...] = jnp.maximum(o, 0.0)

    return pl.pallas_call(
        body,
        grid=(nb,),
        in_specs=[
            pl.BlockSpec((256, 128), lambda i: (i, 0)),
            pl.BlockSpec((256, 128), lambda i, _nb=nb: (_nb + i, 0)),
            pl.BlockSpec((256, 128), lambda i: (i, 0)),
            pl.BlockSpec((128, 128), lambda i: (0, 0)),
            pl.BlockSpec((1, 128), lambda i: (0, 0)),
            pl.BlockSpec((128, 16), lambda i: (0, 0)),
            pl.BlockSpec((1, 16), lambda i: (0, 0)),
        ],
        out_specs=pl.BlockSpec((256, 16), lambda i: (i, 0)),
        out_shape=jax.ShapeDtypeStruct((prow, 16), F32),
    )(parts, parts, hp, rootb, biasb, outb, bntb)


def _sc_gather(tbln, dst2, src2):
    npad = tbln.shape[0]
    epad = dst2.shape[0] * 128
    ept = epad // _NW
    chunks = ept // _C
    mesh = plsc.VectorSubcoreMesh(core_axis_name="c", subcore_axis_name="s")

    cg = 1024
    nch = ept // cg
    rows_per = ept // 128

    @functools.partial(
        pl.kernel,
        out_type=[jax.ShapeDtypeStruct((epad, 16), F32),
                  jax.ShapeDtypeStruct((epad, 16), F32)],
        mesh=mesh,
        scratch_types=[
            pltpu.VMEM((rows_per, 128), jnp.int32),
            pltpu.VMEM((rows_per, 128), jnp.int32),
            pltpu.VMEM((2, cg, 16), F32),
            pltpu.VMEM((2, cg, 16), F32),
            pltpu.SemaphoreType.DMA,
            pltpu.SemaphoreType.DMA,
            pltpu.SemaphoreType.DMA,
        ],
        compiler_params=pltpu.CompilerParams(use_tc_tiling_on_sc=False),
    )
    def k(tbl_hbm, dst_hbm, src_hbm, xd_hbm, xs_hbm,
          idx_d, idx_s, rows_d, rows_s, sem_g, sem_w0, sem_w1):
        cid = lax.axis_index("c")
        sid = lax.axis_index("s")
        wid = cid * 16 + sid
        pltpu.sync_copy(dst_hbm.at[pl.ds(wid * rows_per, rows_per)], idx_d)
        pltpu.sync_copy(src_hbm.at[pl.ds(wid * rows_per, rows_per)], idx_s)
        sem_w = (sem_w0, sem_w1)
        pend = [None, None]
        for j in range(nch):
            slot = j % 2
            if pend[slot] is not None:
                for cp in pend[slot]:
                    cp.wait()
            gcps = []
            for t in range(cg // 128):
                r = j * (cg // 128) + t
                gcps.append(pltpu.async_copy(
                    tbl_hbm.at[idx_d.at[r]],
                    rows_d.at[slot, pl.ds(t * 128, 128)], sem_g))
                gcps.append(pltpu.async_copy(
                    tbl_hbm.at[idx_s.at[r]],
                    rows_s.at[slot, pl.ds(t * 128, 128)], sem_g))
            for cp in gcps:
                cp.wait()
            base = wid * ept + j * cg
            pend[slot] = [
                pltpu.async_copy(rows_d.at[slot],
                                 xd_hbm.at[pl.ds(base, cg)], sem_w[slot]),
                pltpu.async_copy(rows_s.at[slot],
                                 xs_hbm.at[pl.ds(base, cg)], sem_w[slot]),
            ]
        for slot in range(2):
            if pend[slot] is not None:
                for cp in pend[slot]:
                    cp.wait()

    return k(tbln, dst2, src2)


def _sc_scatter(msg128, dst2, ztbl):
    npad = ztbl.shape[0]
    epad = dst2.shape[0] * 128
    ept = epad // _NW
    chunks = ept // _C
    stripe = npad // 16
    mesh = plsc.VectorSubcoreMesh(core_axis_name="c", subcore_axis_name="s")

    @functools.partial(
        pl.kernel,
        out_type=jax.ShapeDtypeStruct((2 * npad, 16), F32),
        mesh=mesh,
        scratch_types=[
            pltpu.VMEM((_C // 128, 128), jnp.int32),
            pltpu.VMEM((_C, 16), F32),
            pltpu.VMEM_SHARED((npad, 16), F32),
        ],
        compiler_params=pltpu.CompilerParams(use_tc_tiling_on_sc=False),
    )
    def k(msg_hbm, dst_hbm, z_hbm, out_hbm, idx_v, rows_v, shared):
        cid = lax.axis_index("c")
        sid = lax.axis_index("s")
        pltpu.sync_copy(z_hbm.at[pl.ds(sid * stripe, stripe)],
                        shared.at[pl.ds(sid * stripe, stripe)])
        plsc.subcore_barrier()
        for j in range(chunks):
            base = (cid * 16 + sid) * ept + j * _C
            pltpu.sync_copy(dst_hbm.at[pl.ds(base // 128, _C // 128)], idx_v)
            pltpu.sync_copy(msg_hbm.at[pl.ds(base, _C)], rows_v)
            for t in range(_C // 128):
                pltpu.sync_copy(rows_v.at[pl.ds(t * 128, 128)],
                                shared.at[idx_v.at[t]], add=True)
        plsc.subcore_barrier()
        pltpu.sync_copy(shared.at[pl.ds(sid * stripe, stripe)],
                        out_hbm.at[pl.ds(cid * npad + sid * stripe, stripe)])

    return k(msg128, dst2, ztbl)


def _conv(hp, dst2, src2, h1p, w2blk, b2blk, expblk, ztbl):
    npad = hp.shape[0] * 8
    epad = dst2.shape[0] * 128
    xd, xs = _sc_gather(hp.reshape(npad, 16), dst2, src2)
    msg128 = _msg_call(h1p, xd.reshape(epad // 8, 128),
                       xs.reshape(epad // 8, 128), w2blk, b2blk, expblk)
    parts = _sc_scatter(msg128.reshape(epad, 16), dst2, ztbl)
    return parts.reshape(npad // 4, 128)


def kernel(x, edge_index, edge_attr, W_ne, b_ne, W_ee, b_ee, We1, be1, We2,
           be2, root_e1, bias_e1, root_e2, bias_e2, Wd1, bd1, Wd2, bd2,
           root_d1, bias_d1, root_d2, bias_d2, W_nt, b_nt):
    n, node_f = x.shape
    e, edge_f = edge_attr.shape

    npad = _cdiv(n, 2048) * 2048
    epad = _cdiv(e, _NW * _C) * (_NW * _C)

    src = edge_index[0]
    dst = edge_index[1]
    src_p = jnp.concatenate([src, jnp.zeros((epad - e,), jnp.int32)])
    dst_p = jnp.concatenate(
        [dst, jnp.full((epad - e,), npad - 1, jnp.int32)])
    src2 = src_p.reshape(epad // 128, 128)
    dst2 = dst_p.reshape(epad // 128, 128)

    xblk = jnp.pad(x, ((0, npad - n), (0, 0))).reshape(npad // 8, 8 * node_f)
    ea128 = jnp.pad(edge_attr.reshape(e // 8, 128),
                    ((0, (epad - e) // 8), (0, 0)))
    ztbl = jnp.zeros((npad, 16), F32)

    j = jnp.arange(256)
    e_j = jnp.where(j < 128, j // 16, (j - 128) // 16)
    c_j = jnp.where(j < 128, j % 16, 16 + (j - 128) % 16)
    col = jnp.arange(4096)
    expblk = ((col[None, :] // 128 == c_j[:, None])
              & ((col[None, :] % 128) // 16 == e_j[:, None])).astype(jnp.bfloat16)

    def _w2com(w2):
        return (_kron8(w2).reshape(256, 8, 32, 16)
                .transpose(0, 2, 1, 3).reshape(256, 4096))

    def _b2com(b2):
        return jnp.broadcast_to(b2.reshape(32, 1, 16),
                                (32, 8, 16)).reshape(1, 4096)

    w2eb, b2eb = _w2com(We2), _b2com(be2)
    w2db, b2db = _w2com(Wd2), _b2com(bd2)

    h0p = _node_enc_call(xblk, _kron8(W_ne), _tile8(b_ne))
    h1ep, eep, h1dp = _edge_prep_call(
        ea128, _kron8(We1), _tile8(be1), _kron8(W_ee), _tile8(b_ee),
        _kron8(Wd1), _tile8(bd1))

    parts = _conv(h0p, dst2, src2, h1ep, w2eb, b2eb, expblk, ztbl)
    hp = _combine_call(parts, h0p, _kron8(root_e1), _tile8(bias_e1))
    parts = _conv(hp, dst2, src2, h1ep, w2eb, b2eb, expblk, ztbl)
    hp = _combine_call(parts, hp, _kron8(root_e2), _tile8(bias_e2))
    parts = _conv(hp, dst2, src2, h1dp, w2db, b2db, expblk, ztbl)
    hp = _combine_call(parts, hp, _kron8(root_d1), _tile8(bias_d1))
    parts = _conv(hp, dst2, src2, h1dp, w2db, b2db, expblk, ztbl)
    outp = _combine_final_call(parts, hp, _kron8(root_d2), _tile8(bias_d2),
                               _kron8(W_nt), _tile8(b_nt))
    return outp.reshape(npad, 2)[:n]
